# baseline (speedup 1.0000x reference)
import functools

import jax
import jax.numpy as jnp
from jax import lax
from jax.experimental import pallas as pl
from jax.experimental.pallas import tpu as pltpu
from jax.experimental.pallas import tpu_sc as plsc

_H = 24
_F = 32
_NC = 2
_NS = 16
_SC_MESH = dict(core_axis_name="c", subcore_axis_name="s")


def _edge_msg(e, hsrc, epw, epb, egw, egb, *, tile=2000, interpret=False):
    E, ED = e.shape

    def body(e_ref, h_ref, w_ref, b_ref, gw_ref, gb_ref, out_ref):
        eb = e_ref[...]
        p0 = jax.lax.dot_general(
            eb.astype(jnp.bfloat16), w_ref[...].astype(jnp.bfloat16),
            (((1,), (0,)), ((), ())), preferred_element_type=jnp.float32)
        proj = jnp.maximum(p0 + b_ref[...], 0.0)
        gate = jax.nn.sigmoid(eb @ gw_ref[...] + gb_ref[...])
        w = proj * gate
        h = h_ref[...]
        acc = h[:, 0:1] * w[:, 0:_H]
        for i in range(1, _H):
            acc = acc + h[:, i : i + 1] * w[:, i * _H : (i + 1) * _H]
        out_ref[...] = jnp.concatenate(
            [acc, jnp.zeros((acc.shape[0], _F - _H), jnp.float32)], axis=1)

    hw = hsrc.shape[1]
    return pl.pallas_call(
        body,
        grid=(E // tile,),
        in_specs=[
            pl.BlockSpec((tile, ED), lambda i: (i, 0)),
            pl.BlockSpec((tile, hw), lambda i: (i, 0)),
            pl.BlockSpec((ED, ED), lambda i: (0, 0)),
            pl.BlockSpec((1, ED), lambda i: (0, 0)),
            pl.BlockSpec((ED, 1), lambda i: (0, 0)),
            pl.BlockSpec((1, 1), lambda i: (0, 0)),
        ],
        out_specs=pl.BlockSpec((tile, _F), lambda i: (i, 0)),
        out_shape=jax.ShapeDtypeStruct((E, _F), jnp.float32),
        interpret=interpret,
    )(e, hsrc, epw, epb.reshape(1, ED), egw, egb.reshape(1, 1))


def _interaction(sf, vf, slen_t, vlen, *, rt=200, interpret=False):
    N1, h = sf.shape
    N2 = vf.shape[0]
    I = N1 // rt

    def body(sf_ref, vf_ref, sl_ref, vl_ref, imap_ref, sp_ref, vp_ref, acc_ref):
        i = pl.program_id(0)
        sfb = sf_ref[...]
        vfb = vf_ref[...]
        t = jnp.tanh(jax.lax.dot_general(sfb, vfb, (((1,), (1,)), ((), ()))))
        t = t * sl_ref[...] * vl_ref[...]
        imap_ref[...] = t
        sp_ref[...] = t @ vfb
        vpc = jax.lax.dot_general(t, sfb, (((0,), (0,)), ((), ())))

        @pl.when(i == 0)
        def _():
            acc_ref[...] = vpc

        @pl.when(i > 0)
        def _():
            acc_ref[...] = acc_ref[...] + vpc

        @pl.when(i == I - 1)
        def _():
            vp_ref[...] = acc_ref[...]

    return pl.pallas_call(
        body,
        grid=(I,),
        in_specs=[
            pl.BlockSpec((rt, h), lambda i: (i, 0)),
            pl.BlockSpec((N2, h), lambda i: (0, 0)),
            pl.BlockSpec((rt, 1), lambda i: (i, 0)),
            pl.BlockSpec((1, N2), lambda i: (0, 0)),
        ],
        out_specs=[
            pl.BlockSpec((rt, N2), lambda i: (i, 0)),
            pl.BlockSpec((rt, h), lambda i: (i, 0)),
            pl.BlockSpec((N2, h), lambda i: (0, 0)),
        ],
        out_shape=[
            jax.ShapeDtypeStruct((N1, N2), jnp.float32),
            jax.ShapeDtypeStruct((N1, h), jnp.float32),
            jax.ShapeDtypeStruct((N2, h), jnp.float32),
        ],
        scratch_shapes=[pltpu.VMEM((N2, h), jnp.float32)],
        interpret=interpret,
    )(sf, vf, slen_t, vlen)


def _sc_rwse(s_src, s_dst, v_src, v_dst, n, k=16):
    e = s_src.shape[0]
    eps = e // _NS
    npad = ((n + _NS * 16 - 1) // (_NS * 16)) * (_NS * 16)
    nps = npad // _NS
    nv = nps // 16

    def graph(cid, src_hbm, dst_hbm, ones_hbm, zer_hbm, out_hbm,
              src_v, dst_v, vals_v, col_v, deg_v, zer_v, pbuf, acc, sem):
        sid = lax.axis_index("s")
        sl = pl.ds(sid * nps, nps)
        pltpu.sync_copy(src_hbm.at[pl.ds(sid * eps, eps)], src_v)
        pltpu.sync_copy(dst_hbm.at[pl.ds(sid * eps, eps)], dst_v)
        pltpu.sync_copy(ones_hbm, vals_v)
        pltpu.sync_copy(zer_hbm, zer_v)
        pltpu.sync_copy(zer_v, acc.at[sl])
        plsc.subcore_barrier()
        pltpu.sync_copy(vals_v, acc.at[dst_v], add=True)
        plsc.subcore_barrier()
        pltpu.sync_copy(acc.at[sl], col_v)

        @pl.loop(0, nv)
        def _(j):
            v = col_v[pl.ds(j * 16, 16)]
            v = jnp.where(v == 0.0, 1.0, v)
            col_v[pl.ds(j * 16, 16)] = v
            deg_v[pl.ds(j * 16, 16)] = v

        pltpu.sync_copy(col_v, pbuf.at[sl])
        pltpu.sync_copy(col_v, out_hbm.at[cid, 0, sl])
        plsc.subcore_barrier()
        for kk in range(1, k):
            pltpu.sync_copy(zer_v, acc.at[sl])
            pltpu.async_copy(pbuf.at[src_v], vals_v, sem).wait()
            plsc.subcore_barrier()
            pltpu.sync_copy(vals_v, acc.at[dst_v], add=True)
            plsc.subcore_barrier()
            pltpu.sync_copy(acc.at[sl], col_v)

            @pl.loop(0, nv)
            def _(j):
                v = col_v[pl.ds(j * 16, 16)] / deg_v[pl.ds(j * 16, 16)]
                col_v[pl.ds(j * 16, 16)] = v

            pltpu.sync_copy(col_v, pbuf.at[sl])
            pltpu.sync_copy(col_v, out_hbm.at[cid, kk, sl])
            plsc.subcore_barrier()

    @functools.partial(
        pl.kernel,
        out_type=jax.ShapeDtypeStruct((2, k, npad), jnp.float32),
        mesh=plsc.VectorSubcoreMesh(**_SC_MESH),
        scratch_types=[
            pltpu.VMEM((eps,), jnp.int32),
            pltpu.VMEM((eps,), jnp.int32),
            pltpu.VMEM((eps,), jnp.float32),
            pltpu.VMEM((nps,), jnp.float32),
            pltpu.VMEM((nps,), jnp.float32),
            pltpu.VMEM((nps,), jnp.float32),
            pltpu.VMEM_SHARED((npad,), jnp.float32),
            pltpu.VMEM_SHARED((npad,), jnp.float32),
            pltpu.SemaphoreType.DMA,
        ],
    )
    def run(ss_hbm, sd_hbm, vs_hbm, vd_hbm, ones_hbm, zer_hbm, out_hbm,
            src_v, dst_v, vals_v, col_v, deg_v, zer_v, pbuf, acc, sem):
        cid = lax.axis_index("c")

        @pl.when(cid == 0)
        def _():
            graph(0, ss_hbm, sd_hbm, ones_hbm, zer_hbm, out_hbm,
                  src_v, dst_v, vals_v, col_v, deg_v, zer_v, pbuf, acc, sem)

        @pl.when(cid == 1)
        def _():
            graph(1, vs_hbm, vd_hbm, ones_hbm, zer_hbm, out_hbm,
                  src_v, dst_v, vals_v, col_v, deg_v, zer_v, pbuf, acc, sem)

    ones = jnp.ones((eps,), jnp.float32)
    zer = jnp.zeros((nps,), jnp.float32)
    out = run(s_src, s_dst, v_src, v_dst, ones, zer)
    rwse_s = out[0, :, :n].T
    rwse_v = out[1, :, :n].T
    return rwse_s, rwse_v


def _sc_gather(table, idx, *, chunk=1000):
    n, f = table.shape
    e = idx.shape[0]
    epw = e // (_NC * _NS)
    nch = epw // chunk

    @functools.partial(
        pl.kernel,
        out_type=jax.ShapeDtypeStruct((e, f), jnp.float32),
        mesh=plsc.VectorSubcoreMesh(**_SC_MESH),
        compiler_params=pltpu.CompilerParams(use_tc_tiling_on_sc=False),
        scratch_types=[
            pltpu.VMEM((chunk,), jnp.int32),
            pltpu.VMEM((chunk, f), jnp.float32),
            pltpu.SemaphoreType.DMA,
        ],
    )
    def run(table_hbm, idx_hbm, out_hbm, idx_v, rows_v, sem):
        wid = lax.axis_index("s") * _NC + lax.axis_index("c")
        for ch in range(nch):
            base = wid * epw + ch * chunk
            pltpu.sync_copy(idx_hbm.at[pl.ds(base, chunk)], idx_v)
            pltpu.async_copy(table_hbm.at[idx_v], rows_v, sem).wait()
            pltpu.sync_copy(rows_v, out_hbm.at[pl.ds(base, chunk)])

    return run(table, idx)


def _sc_scatter_sum(rows, dst, n, *, chunk=1000):
    f = rows.shape[1]
    e = dst.shape[0]
    epc = e // _NC
    eps = epc // _NS
    nch = eps // chunk
    npad = ((n + _NS * 16 - 1) // (_NS * 16)) * (_NS * 16)
    nps = npad // _NS

    @functools.partial(
        pl.kernel,
        out_type=jax.ShapeDtypeStruct((_NC, npad, f), jnp.float32),
        mesh=plsc.VectorSubcoreMesh(**_SC_MESH),
        compiler_params=pltpu.CompilerParams(use_tc_tiling_on_sc=False),
        scratch_types=[
            pltpu.VMEM((chunk,), jnp.int32),
            pltpu.VMEM((chunk, f), jnp.float32),
            pltpu.VMEM((nps, f), jnp.float32),
            pltpu.VMEM_SHARED((npad, f), jnp.float32),
            pltpu.SemaphoreType.DMA,
        ],
    )
    def run(rows_hbm, dst_hbm, zer_hbm, out_hbm, didx_v, rows_v, zer_v, acc,
            sem):
        cid = lax.axis_index("c")
        sid = lax.axis_index("s")
        sl = pl.ds(sid * nps, nps)
        pltpu.sync_copy(zer_hbm, zer_v)
        pltpu.sync_copy(zer_v, acc.at[sl])
        plsc.subcore_barrier()
        for ch in range(nch):
            base = cid * epc + sid * eps + ch * chunk
            pltpu.sync_copy(dst_hbm.at[pl.ds(base, chunk)], didx_v)
            pltpu.sync_copy(rows_hbm.at[pl.ds(base, chunk)], rows_v)
            pltpu.sync_copy(rows_v, acc.at[didx_v], add=True)
        plsc.subcore_barrier()
        pltpu.sync_copy(acc.at[sl], zer_v)
        pltpu.sync_copy(zer_v, out_hbm.at[cid, sl])

    zer = jnp.zeros((nps, f), jnp.float32)
    return run(rows, dst, zer)


def _sc_grp2(tab_s, src_s, dst_s, tab_v, src_v, dst_v, n, *, chunk=1000):
    f = tab_s.shape[1]
    e = src_s.shape[0]
    eps = e // _NS
    nch = eps // chunk
    npad = ((n + _NS * 16 - 1) // (_NS * 16)) * (_NS * 16)
    nps = npad // _NS

    def graph(cid, tab_hbm, src_hbm, dst_hbm, zer_hbm, out_hbm,
              didx_v, sidx_v, rows_v, zer_v, acc, sem):
        sid = lax.axis_index("s")
        sl = pl.ds(sid * nps, nps)
        pltpu.sync_copy(zer_hbm, zer_v)
        pltpu.sync_copy(zer_v, acc.at[sl])
        plsc.subcore_barrier()
        for ch in range(nch):
            base = sid * eps + ch * chunk
            pltpu.sync_copy(dst_hbm.at[pl.ds(base, chunk)], didx_v)
            pltpu.sync_copy(src_hbm.at[pl.ds(base, chunk)], sidx_v)
            pltpu.async_copy(tab_hbm.at[sidx_v], rows_v, sem).wait()
            pltpu.sync_copy(rows_v, acc.at[didx_v], add=True)
        plsc.subcore_barrier()
        pltpu.sync_copy(acc.at[sl], zer_v)
        pltpu.sync_copy(zer_v, out_hbm.at[cid, sl])

    @functools.partial(
        pl.kernel,
        out_type=jax.ShapeDtypeStruct((2, npad, f), jnp.float32),
        mesh=plsc.VectorSubcoreMesh(**_SC_MESH),
        compiler_params=pltpu.CompilerParams(use_tc_tiling_on_sc=False),
        scratch_types=[
            pltpu.VMEM((chunk,), jnp.int32),
            pltpu.VMEM((chunk,), jnp.int32),
            pltpu.VMEM((chunk, f), jnp.float32),
            pltpu.VMEM((nps, f), jnp.float32),
            pltpu.VMEM_SHARED((npad, f), jnp.float32),
            pltpu.SemaphoreType.DMA,
        ],
    )
    def run(ts_hbm, ss_hbm, sd_hbm, tv_hbm, vs_hbm, vd_hbm, zer_hbm, out_hbm,
            didx_v, sidx_v, rows_v, zer_v, acc, sem):
        cid = lax.axis_index("c")

        @pl.when(cid == 0)
        def _():
            graph(0, ts_hbm, ss_hbm, sd_hbm, zer_hbm, out_hbm,
                  didx_v, sidx_v, rows_v, zer_v, acc, sem)

        @pl.when(cid == 1)
        def _():
            graph(1, tv_hbm, vs_hbm, vd_hbm, zer_hbm, out_hbm,
                  didx_v, sidx_v, rows_v, zer_v, acc, sem)

    zer = jnp.zeros((nps, f), jnp.float32)
    return run(tab_s, src_s, dst_s, tab_v, src_v, dst_v, zer)


def _node1(x, rwse, w, b, *, tile=2000):
    n = x.shape[0]

    def body(x_ref, r_ref, w_ref, b_ref, out_ref):
        nf = jnp.concatenate([x_ref[...], r_ref[...]], axis=1)
        h0 = jnp.maximum(jnp.dot(nf, w_ref[...], precision=jax.lax.Precision.HIGHEST) + b_ref[...], 0.0)
        out_ref[...] = jnp.concatenate(
            [h0, jnp.zeros((tile, _F - _H), jnp.float32)], axis=1)

    return pl.pallas_call(
        body,
        grid=(n // tile,),
        in_specs=[
            pl.BlockSpec((tile, x.shape[1]), lambda i: (i, 0)),
            pl.BlockSpec((tile, rwse.shape[1]), lambda i: (i, 0)),
            pl.BlockSpec(w.shape, lambda i: (0, 0)),
            pl.BlockSpec((1, _H), lambda i: (0, 0)),
        ],
        out_specs=pl.BlockSpec((tile, _F), lambda i: (i, 0)),
        out_shape=jax.ShapeDtypeStruct((n, _F), jnp.float32),
    )(x, rwse, w, b.reshape(1, _H))


def _node2(h0, rwse, agg2, conv_b, mw, mb, *, tile=2000):
    n = h0.shape[0]
    has_agg = agg2 is not None

    def body(*refs):
        if has_agg:
            h_ref, r_ref, a_ref, cb_ref, w_ref, b_ref, out_ref = refs
            agg = (a_ref[0] + a_ref[1])[:, :_H]
            deg = r_ref[...][:, 0:1]
            m = jnp.maximum(agg / deg + h_ref[...][:, :_H] + cb_ref[...], 0.0)
        else:
            h_ref, cb_ref, w_ref, b_ref, out_ref = refs
            m = jnp.maximum(h_ref[...][:, :_H] + cb_ref[...], 0.0)
        cat = jnp.concatenate([m, h_ref[...][:, :_H]], axis=1)
        o = jnp.dot(cat, w_ref[...], precision=jax.lax.Precision.HIGHEST) + b_ref[...]
        out_ref[...] = jnp.concatenate(
            [o, jnp.zeros((tile, _F - _H), jnp.float32)], axis=1)

    in_specs = [pl.BlockSpec((tile, _F), lambda i: (i, 0))]
    args = [h0]
    if has_agg:
        in_specs += [
            pl.BlockSpec((tile, rwse.shape[1]), lambda i: (i, 0)),
            pl.BlockSpec((2, tile, _F), lambda i: (0, i, 0)),
        ]
        args += [rwse, agg2]
    in_specs += [
        pl.BlockSpec((1, _H), lambda i: (0, 0)),
        pl.BlockSpec(mw.shape, lambda i: (0, 0)),
        pl.BlockSpec((1, _H), lambda i: (0, 0)),
    ]
    args += [conv_b.reshape(1, _H), mw, mb.reshape(1, _H)]
    return pl.pallas_call(
        body,
        grid=(n // tile,),
        in_specs=in_specs,
        out_specs=pl.BlockSpec((tile, _F), lambda i: (i, 0)),
        out_shape=jax.ShapeDtypeStruct((n, _F), jnp.float32),
    )(*args)


def _node3(out1, rwse, grp2, x, sw, sb, gidx, *, tile=2000):
    n = out1.shape[0]

    def body(o_ref, r_ref, g_ref, x_ref, w_ref, b_ref, out_ref):
        r = r_ref[...]
        deg = r[:, 0:1]
        grp_m = g_ref[0][:, :_H] / deg
        cat = jnp.concatenate([o_ref[...][:, :_H], grp_m], axis=1)
        o = jnp.dot(cat, w_ref[...], precision=jax.lax.Precision.HIGHEST) + b_ref[...]
        out_ref[...] = o + jnp.concatenate([x_ref[...], r], axis=1)

    return pl.pallas_call(
        body,
        grid=(n // tile,),
        in_specs=[
            pl.BlockSpec((tile, _F), lambda i: (i, 0)),
            pl.BlockSpec((tile, rwse.shape[1]), lambda i: (i, 0)),
            pl.BlockSpec((1, tile, _F), lambda i: (gidx, i, 0)),
            pl.BlockSpec((tile, x.shape[1]), lambda i: (i, 0)),
            pl.BlockSpec(sw.shape, lambda i: (0, 0)),
            pl.BlockSpec((1, _H), lambda i: (0, 0)),
        ],
        out_specs=pl.BlockSpec((tile, _H), lambda i: (i, 0)),
        out_shape=jax.ShapeDtypeStruct((n, _H), jnp.float32),
    )(out1, rwse, grp2, x, sw, sb.reshape(1, _H))


def _s2s(fa, fb, p, n_iters=2):
    n = fa.shape[0]
    d = 2 * _H
    wih, whh = p["w_ih"], p["w_hh"]
    bih = p["b_ih"].reshape(1, 4 * d)
    bhh = p["b_hh"].reshape(1, 4 * d)

    def body(fa_ref, fb_ref, wih_ref, whh_ref, bih_ref, bhh_ref, out_ref):
        feat = jnp.concatenate([fa_ref[...], fb_ref[...]], axis=1)
        hh = jnp.zeros((1, d), jnp.float32)
        cc = jnp.zeros((1, d), jnp.float32)
        q_star = jnp.zeros((1, 2 * d), jnp.float32)
        for _ in range(n_iters):
            gates = (
                jax.lax.dot_general(q_star, wih_ref[...], (((1,), (1,)), ((), ())), precision=jax.lax.Precision.HIGHEST)
                + bih_ref[...]
                + jax.lax.dot_general(hh, whh_ref[...], (((1,), (1,)), ((), ())), precision=jax.lax.Precision.HIGHEST)
                + bhh_ref[...]
            )
            gi = gates[:, 0:d]
            gf = gates[:, d:2 * d]
            gg = gates[:, 2 * d:3 * d]
            go = gates[:, 3 * d:4 * d]
            cc = jax.nn.sigmoid(gf) * cc + jax.nn.sigmoid(gi) * jnp.tanh(gg)
            hh = jax.nn.sigmoid(go) * jnp.tanh(cc)
            e = jnp.sum(feat * hh, axis=1, keepdims=True)
            e = e - jnp.max(e, axis=0, keepdims=True)
            ex = jnp.exp(e)
            alpha = ex / jnp.sum(ex, axis=0, keepdims=True)
            readout = jnp.sum(feat * alpha, axis=0, keepdims=True)
            q_star = jnp.concatenate([hh, readout], axis=1)
        mean_feat = jnp.sum(feat, axis=0, keepdims=True) * (1.0 / n)
        out_ref[...] = jnp.concatenate([q_star, mean_feat], axis=1)

    return pl.pallas_call(
        body,
        in_specs=[
            pl.BlockSpec(fa.shape, lambda: (0, 0)),
            pl.BlockSpec(fb.shape, lambda: (0, 0)),
            pl.BlockSpec(wih.shape, lambda: (0, 0)),
            pl.BlockSpec(whh.shape, lambda: (0, 0)),
            pl.BlockSpec((1, 4 * d), lambda: (0, 0)),
            pl.BlockSpec((1, 4 * d), lambda: (0, 0)),
        ],
        out_specs=pl.BlockSpec((1, 3 * d), lambda: (0, 0)),
        out_shape=jax.ShapeDtypeStruct((1, 3 * d), jnp.float32),
    )(fa, fb, wih, whh, bih, bhh)


def _head(ps, pv, p):
    hp = jax.lax.Precision.DEFAULT
    half = ps.shape[1]

    def body(ps_ref, pv_ref, w1, b1, w2, b2, w3, b3, a1, ab1, a2, ab2,
             main_ref, aux_ref):
        psv = ps_ref[...]
        pvv = pv_ref[...]
        h1 = jnp.maximum(
            jnp.dot(psv, w1[0:half, :], precision=hp)
            + jnp.dot(pvv, w1[half:, :], precision=hp) + b1[...], 0.0)
        h2 = jnp.maximum(jnp.dot(h1, w2[...], precision=hp) + b2[...], 0.0)
        main_ref[...] = jnp.sum(h2 * w3[...][:, 0], axis=1, keepdims=True) + b3[...]
        ha = jnp.maximum(
            jnp.dot(psv, a1[0:half, :], precision=hp)
            + jnp.dot(pvv, a1[half:, :], precision=hp) + ab1[...], 0.0)
        aux_ref[...] = jnp.dot(ha, a2[...], precision=hp) + ab2[...]

    full = lambda s: pl.BlockSpec(s, lambda: tuple(0 for _ in s))
    args = [ps, pv,
            p["fc1_w"], p["fc1_b"].reshape(1, -1),
            p["fc2_w"], p["fc2_b"].reshape(1, -1),
            p["fc3_w"], p["fc3_b"].reshape(1, -1),
            p["aux1_w"], p["aux1_b"].reshape(1, -1),
            p["aux2_w"], p["aux2_b"].reshape(1, -1)]
    return pl.pallas_call(
        body,
        in_specs=[full(a.shape) for a in args],
        out_specs=[full((1, 1)), full((1, 3))],
        out_shape=[jax.ShapeDtypeStruct((1, 1), jnp.float32),
                   jax.ShapeDtypeStruct((1, 3), jnp.float32)],
    )(*args)


def kernel(solute_x, solute_edge_index, solute_e, solvent_x, solvent_edge_index,
           solute_len, solvent_len, params):
    s_src, s_dst = solute_edge_index[0], solute_edge_index[1]
    v_src, v_dst = solvent_edge_index[0], solvent_edge_index[1]
    n1 = solute_x.shape[0]
    ps_, pv_ = params["solute"], params["solvent"]

    rwse_s, rwse_v = _sc_rwse(s_src, s_dst, v_src, v_dst, n1)

    h0s = _node1(solute_x, rwse_s, ps_["lin0_w"], ps_["lin0_b"])
    h0v = _node1(solvent_x, rwse_v, pv_["lin0_w"], pv_["lin0_b"])

    hsrc = _sc_gather(h0s, s_src)
    msg = _edge_msg(solute_e, hsrc, ps_["ep_w"], ps_["ep_b"], ps_["eg_w"],
                    ps_["eg_b"])
    agg2 = _sc_scatter_sum(msg, s_dst, n1)

    out1s = _node2(h0s, rwse_s, agg2, ps_["conv_b"], ps_["msg_w"],
                   ps_["msg_b"])
    out1v = _node2(h0v, None, None, pv_["conv_b"], pv_["msg_w"], pv_["msg_b"])

    grp2 = _sc_grp2(out1s, s_src, s_dst, out1v, v_src, v_dst, n1)

    sf = _node3(out1s, rwse_s, grp2, solute_x, ps_["sub_w"], ps_["sub_b"], 0)
    vf = _node3(out1v, rwse_v, grp2, solvent_x, pv_["sub_w"], pv_["sub_b"], 1)

    imap, sp, vp = _interaction(sf, vf, solute_len.T, solvent_len)

    ps = _s2s(sf, sp, params["s2s_solute"])
    pv = _s2s(vf, vp, params["s2s_solvent"])
    main, aux = _head(ps, pv, params)
    return main, aux, imap
